# Initial kernel scaffold; baseline (speedup 1.0000x reference)
import jax
import jax.numpy as jnp
from jax.experimental import pallas as pl


def kernel(x, edge_index, batch, struc, params):
    raise NotImplementedError("write your pallas kernel here")


# speedup vs baseline: 3.1356x; 3.1356x over previous
import jax
import jax.numpy as jnp
from jax import lax
from jax.experimental import pallas as pl
from jax.experimental.pallas import tpu as pltpu
from jax.experimental.pallas import tpu_sc as plsc

N_NODES = 10000
D_FEAT = 128
INFO_DIM = 8
HID = 64
TGT = 10
N_GRAPHS = 128
EPS_BN = 1e-5

R_PAD = 10080
NB = 1008
GRID = R_PAD // NB
PAD_ROW = R_PAD - 1
HPAD = 80

NC = 2
NS = 16
NW = NC * NS
ROWS_PER_TILE = R_PAD // NS

CHUNK = 128
N_EDGES = 320000
CHUNKS = -(-N_EDGES // (NW * CHUNK))
E_PAD = NW * CHUNKS * CHUNK


def _fold_bn(Wp, bp, bn):
    s = bn['g'] / jnp.sqrt(bn['rv'] + EPS_BN)
    Wf = Wp * s[None, :]
    cf = (bp - bn['rm']) * s + bn['b']
    return Wf, cf


def _rep_body(batch_ref, struc_ref, rep_ref):
    b_col = batch_ref[...]
    ids_r = lax.broadcasted_iota(jnp.int32, (NB, N_GRAPHS), 1)
    P = jnp.where(b_col == ids_r, 1.0, 0.0).astype(jnp.float32)
    rep_ref[...] = jnp.dot(P, struc_ref[...], preferred_element_type=jnp.float32)


def _rep_call(batch_col, struc):
    return pl.pallas_call(
        _rep_body,
        grid=(GRID,),
        in_specs=[pl.BlockSpec((NB, 1), lambda i: (i, 0)),
                  pl.BlockSpec((N_GRAPHS, INFO_DIM), lambda i: (0, 0))],
        out_specs=pl.BlockSpec((NB, INFO_DIM), lambda i: (i, 0)),
        out_shape=jax.ShapeDtypeStruct((R_PAD, INFO_DIM), jnp.float32),
    )(batch_col, struc)


def _layer0_body(x_ref, rep_ref, batch_ref, w1x_ref, w1r_ref, c1_ref,
                 w2_ref, c2_ref, l0e_ref, out0_ref, hin_ref):
    pi = pl.program_id(0)
    x = x_ref[...]
    b_col = batch_ref[...]
    rep = rep_ref[...]
    t = jnp.dot(x, w1x_ref[...], preferred_element_type=jnp.float32)
    t += jnp.dot(rep, w1r_ref[...], preferred_element_type=jnp.float32)
    t = jnp.maximum(t + c1_ref[...], 0.0)
    h = jnp.maximum(jnp.dot(t, w2_ref[...], preferred_element_type=jnp.float32)
                    + c2_ref[...], 0.0)
    h = jnp.where(b_col >= 0, h, 0.0)
    mask = jnp.where(b_col >= 0, 1.0, 0.0).astype(jnp.float32)
    v = jnp.concatenate(
        [h, rep, mask, jnp.zeros((NB, HPAD - HID - INFO_DIM - 1), jnp.float32)],
        axis=1)
    ids_c = lax.broadcasted_iota(jnp.int32, (N_GRAPHS, NB), 0)
    PT = jnp.where(b_col.reshape(1, NB) == ids_c, 1.0, 0.0).astype(jnp.float32)
    pooled = jnp.dot(PT, v, preferred_element_type=jnp.float32)

    @pl.when(pi == 0)
    def _():
        out0_ref[...] = jnp.zeros((N_GRAPHS, TGT), jnp.float32)

    out0_ref[...] += jnp.dot(pooled, l0e_ref[...],
                             preferred_element_type=jnp.float32)
    hin_ref[...] = v


def _layer0_call(x_pad, rep, batch_col, w1x, w1r, c1, w2, c2, l0e):
    full = lambda shape: pl.BlockSpec(shape, lambda i: (0,) * len(shape))
    return pl.pallas_call(
        _layer0_body,
        grid=(GRID,),
        in_specs=[
            pl.BlockSpec((NB, D_FEAT), lambda i: (i, 0)),
            pl.BlockSpec((NB, INFO_DIM), lambda i: (i, 0)),
            pl.BlockSpec((NB, 1), lambda i: (i, 0)),
            full((D_FEAT, HID)), full((INFO_DIM, HID)), full((1, HID)),
            full((HID, HID)), full((1, HID)),
            full((HPAD, TGT)),
        ],
        out_specs=(pl.BlockSpec((N_GRAPHS, TGT), lambda i: (0, 0)),
                   pl.BlockSpec((NB, HPAD), lambda i: (i, 0))),
        out_shape=(jax.ShapeDtypeStruct((N_GRAPHS, TGT), jnp.float32),
                   jax.ShapeDtypeStruct((R_PAD, HPAD), jnp.float32)),
    )(x_pad, rep, batch_col, w1x, w1r, c1, w2, c2, l0e)


def _layer_body(hin_ref, agg2_ref, batch_ref, struc_ref,
                w1_ref, c1_ref, w2_ref, c2_ref, lh_ref, lr_ref, bl_ref,
                outc_ref, hin_out_ref):
    pi = pl.program_id(0)
    u = hin_ref[...] + agg2_ref[0] + agg2_ref[1]
    t = jnp.maximum(jnp.dot(u, w1_ref[...], preferred_element_type=jnp.float32)
                    + c1_ref[...], 0.0)
    h = jnp.maximum(jnp.dot(t, w2_ref[...], preferred_element_type=jnp.float32)
                    + c2_ref[...], 0.0)
    b_col = batch_ref[...]
    h = jnp.where(b_col >= 0, h, 0.0)
    ids_c = lax.broadcasted_iota(jnp.int32, (N_GRAPHS, NB), 0)
    PT = jnp.where(b_col.reshape(1, NB) == ids_c, 1.0, 0.0).astype(jnp.float32)
    pooled = jnp.dot(PT, h, preferred_element_type=jnp.float32)

    @pl.when(pi == 0)
    def _():
        outc_ref[...] = (jnp.dot(struc_ref[...], lr_ref[...],
                                 preferred_element_type=jnp.float32)
                         + bl_ref[...])

    outc_ref[...] += jnp.dot(pooled, lh_ref[...],
                             preferred_element_type=jnp.float32)
    hin_out_ref[...] = jnp.zeros((NB, HPAD), jnp.float32)
    hin_out_ref[:, 0:HID] = h
    hin_out_ref[:, HID:HID + INFO_DIM] = hin_ref[:, HID:HID + INFO_DIM]


def _layer_call(hin, agg2, batch_col, struc, w1, c1, w2, c2, lh, lr, bl):
    full = lambda shape: pl.BlockSpec(shape, lambda i: (0,) * len(shape))
    return pl.pallas_call(
        _layer_body,
        grid=(GRID,),
        in_specs=[
            pl.BlockSpec((NB, HPAD), lambda i: (i, 0)),
            pl.BlockSpec((NC, NB, HPAD), lambda i: (0, i, 0)),
            pl.BlockSpec((NB, 1), lambda i: (i, 0)),
            full((N_GRAPHS, INFO_DIM)),
            full((HPAD, HID)), full((1, HID)),
            full((HID, HID)), full((1, HID)),
            full((HID, TGT)), full((INFO_DIM, TGT)), full((1, TGT)),
        ],
        out_specs=(pl.BlockSpec((N_GRAPHS, TGT), lambda i: (0, 0)),
                   pl.BlockSpec((NB, HPAD), lambda i: (i, 0))),
        out_shape=(jax.ShapeDtypeStruct((N_GRAPHS, TGT), jnp.float32),
                   jax.ShapeDtypeStruct((R_PAD, HPAD), jnp.float32)),
    )(hin, agg2, batch_col, struc, w1, c1, w2, c2, lh, lr, bl)


def _sc_agg_body(hin_hbm, src_hbm, dst_hbm, zeros_hbm, out_hbm,
                 src_vm, dst_vm, rows_vm, spmem, sem):
    c = lax.axis_index("c")
    s = lax.axis_index("s")
    wid = c * NS + s
    pltpu.sync_copy(zeros_hbm.at[pl.ds(s * ROWS_PER_TILE, ROWS_PER_TILE)],
                    spmem.at[pl.ds(s * ROWS_PER_TILE, ROWS_PER_TILE)])
    pltpu.sync_copy(src_hbm.at[wid], src_vm)
    pltpu.sync_copy(dst_hbm.at[wid], dst_vm)
    plsc.subcore_barrier()

    def body(j, carry):
        pltpu.async_copy(hin_hbm.at[src_vm.at[j]], rows_vm, sem).wait()
        pltpu.sync_copy(rows_vm, spmem.at[dst_vm.at[j]], add=True)
        return carry

    lax.fori_loop(0, CHUNKS, body, 0)
    plsc.subcore_barrier()
    pltpu.sync_copy(spmem.at[pl.ds(s * ROWS_PER_TILE, ROWS_PER_TILE)],
                    out_hbm.at[c, pl.ds(s * ROWS_PER_TILE, ROWS_PER_TILE)])


def _sc_agg(hin, src3d, dst3d, zeros):
    mesh = plsc.VectorSubcoreMesh(core_axis_name="c", subcore_axis_name="s")
    return pl.kernel(
        _sc_agg_body,
        out_type=jax.ShapeDtypeStruct((NC, R_PAD, HPAD), jnp.float32),
        mesh=mesh,
        compiler_params=pltpu.CompilerParams(use_tc_tiling_on_sc=False),
        scratch_types=[
            pltpu.VMEM((CHUNKS, CHUNK), jnp.int32),
            pltpu.VMEM((CHUNKS, CHUNK), jnp.int32),
            pltpu.VMEM((CHUNK, HPAD), jnp.float32),
            pltpu.VMEM_SHARED((R_PAD, HPAD), jnp.float32),
            pltpu.SemaphoreType.DMA,
        ],
    )(hin, src3d, dst3d, zeros)


def kernel(x, edge_index, batch, struc, params):
    p0 = params['mlp0']
    W1f, c1 = _fold_bn(p0['W1'], p0['b1'], p0['bn1'])
    W2f, c2 = _fold_bn(p0['W2'], p0['b2'], p0['bn2'])
    w0 = (W1f[:D_FEAT], W1f[D_FEAT:], c1.reshape(1, HID),
          W2f, c2.reshape(1, HID))
    wl = {}
    for l in range(1, 5):
        pl_ = params['mlp%d' % l]
        W1f, c1 = _fold_bn(pl_['W1'], pl_['b1'], pl_['bn1'])
        W2f, c2 = _fold_bn(pl_['W2'], pl_['b2'], pl_['bn2'])
        W1e = jnp.zeros((HPAD, HID), jnp.float32).at[:HID + INFO_DIM].set(W1f)
        wl[l] = (W1e, c1.reshape(1, HID), W2f, c2.reshape(1, HID))
    lin = {}
    for l in range(5):
        lp = params['lin%d' % l]
        lin[l] = (lp['W'][:HID], lp['W'][HID:], lp['b'].reshape(1, TGT))

    x_pad = jnp.concatenate(
        [x, jnp.zeros((R_PAD - N_NODES, D_FEAT), jnp.float32)])
    batch_col = jnp.concatenate(
        [batch, jnp.full((R_PAD - N_NODES,), -1, jnp.int32)]).reshape(R_PAD, 1)

    src = edge_index[0]
    dst = edge_index[1]
    pad = E_PAD - N_EDGES
    src3d = jnp.concatenate(
        [src, jnp.full((pad,), PAD_ROW, jnp.int32)]).reshape(NW, CHUNKS, CHUNK)
    dst3d = jnp.concatenate(
        [dst, jnp.full((pad,), PAD_ROW, jnp.int32)]).reshape(NW, CHUNKS, CHUNK)
    zeros = jnp.zeros((R_PAD, HPAD), jnp.float32)

    rep = _rep_call(batch_col, struc)
    w1x, w1r, c1, w2, c2 = w0
    l0h, l0r, b0 = lin[0]
    l0e = jnp.concatenate(
        [l0h, l0r, b0, jnp.zeros((HPAD - HID - INFO_DIM - 1, TGT), jnp.float32)])
    out, hin = _layer0_call(x_pad, rep, batch_col, w1x, w1r, c1, w2, c2, l0e)

    for l in range(1, 5):
        agg2 = _sc_agg(hin, src3d, dst3d, zeros)
        w1, c1, w2, c2 = wl[l]
        lh, lr, bl = lin[l]
        outc, hin = _layer_call(hin, agg2, batch_col, struc,
                                w1, c1, w2, c2, lh, lr, bl)
        out = out + outc
    return out


# speedup vs baseline: 3.5816x; 1.1422x over previous
import jax
import jax.numpy as jnp
from jax import lax
from jax.experimental import pallas as pl
from jax.experimental.pallas import tpu as pltpu
from jax.experimental.pallas import tpu_sc as plsc

N_NODES = 10000
D_FEAT = 128
INFO_DIM = 8
HID = 64
TGT = 10
N_GRAPHS = 128
EPS_BN = 1e-5

R_PAD = 10080
NB = 1008
GRID = R_PAD // NB
PAD_ROW = R_PAD - 1
HPAD = 80

NC = 2
NS = 16
NW = NC * NS
ROWS_PER_TILE = R_PAD // NS

CHUNK = 128
N_EDGES = 320000
CHUNKS = -(-N_EDGES // (NW * CHUNK))
E_PAD = NW * CHUNKS * CHUNK


def _fold_bn(Wp, bp, bn):
    s = bn['g'] / jnp.sqrt(bn['rv'] + EPS_BN)
    Wf = Wp * s[None, :]
    cf = (bp - bn['rm']) * s + bn['b']
    return Wf, cf


def _rep_body(batch_ref, struc_ref, rep_ref):
    b_col = batch_ref[...]
    ids_r = lax.broadcasted_iota(jnp.int32, (NB, N_GRAPHS), 1)
    P = jnp.where(b_col == ids_r, 1.0, 0.0).astype(jnp.float32)
    rep_ref[...] = jnp.dot(P, struc_ref[...], preferred_element_type=jnp.float32)


def _rep_call(batch_col, struc):
    return pl.pallas_call(
        _rep_body,
        grid=(GRID,),
        in_specs=[pl.BlockSpec((NB, 1), lambda i: (i, 0)),
                  pl.BlockSpec((N_GRAPHS, INFO_DIM), lambda i: (0, 0))],
        out_specs=pl.BlockSpec((NB, INFO_DIM), lambda i: (i, 0)),
        out_shape=jax.ShapeDtypeStruct((R_PAD, INFO_DIM), jnp.float32),
    )(batch_col, struc)


def _layer0_body(x_ref, rep_ref, batch_ref, w1x_ref, w1r_ref, c1_ref,
                 w2_ref, c2_ref, l0e_ref, out0_ref, hin_ref):
    pi = pl.program_id(0)
    x = x_ref[...]
    b_col = batch_ref[...]
    rep = rep_ref[...]
    t = jnp.dot(x, w1x_ref[...], preferred_element_type=jnp.float32)
    t += jnp.dot(rep, w1r_ref[...], preferred_element_type=jnp.float32)
    t = jnp.maximum(t + c1_ref[...], 0.0)
    h = jnp.maximum(jnp.dot(t, w2_ref[...], preferred_element_type=jnp.float32)
                    + c2_ref[...], 0.0)
    h = jnp.where(b_col >= 0, h, 0.0)
    mask = jnp.where(b_col >= 0, 1.0, 0.0).astype(jnp.float32)
    v = jnp.concatenate(
        [h, rep, mask, jnp.zeros((NB, HPAD - HID - INFO_DIM - 1), jnp.float32)],
        axis=1)
    ids_c = lax.broadcasted_iota(jnp.int32, (N_GRAPHS, NB), 0)
    PT = jnp.where(b_col.reshape(1, NB) == ids_c, 1.0, 0.0).astype(jnp.float32)
    pooled = jnp.dot(PT, v, preferred_element_type=jnp.float32)

    @pl.when(pi == 0)
    def _():
        out0_ref[...] = jnp.zeros((N_GRAPHS, TGT), jnp.float32)

    out0_ref[...] += jnp.dot(pooled, l0e_ref[...],
                             preferred_element_type=jnp.float32)
    hin_ref[...] = v


def _layer0_call(x_pad, rep, batch_col, w1x, w1r, c1, w2, c2, l0e):
    full = lambda shape: pl.BlockSpec(shape, lambda i: (0,) * len(shape))
    return pl.pallas_call(
        _layer0_body,
        grid=(GRID,),
        in_specs=[
            pl.BlockSpec((NB, D_FEAT), lambda i: (i, 0)),
            pl.BlockSpec((NB, INFO_DIM), lambda i: (i, 0)),
            pl.BlockSpec((NB, 1), lambda i: (i, 0)),
            full((D_FEAT, HID)), full((INFO_DIM, HID)), full((1, HID)),
            full((HID, HID)), full((1, HID)),
            full((HPAD, TGT)),
        ],
        out_specs=(pl.BlockSpec((N_GRAPHS, TGT), lambda i: (0, 0)),
                   pl.BlockSpec((NB, HPAD), lambda i: (i, 0))),
        out_shape=(jax.ShapeDtypeStruct((N_GRAPHS, TGT), jnp.float32),
                   jax.ShapeDtypeStruct((R_PAD, HPAD), jnp.float32)),
    )(x_pad, rep, batch_col, w1x, w1r, c1, w2, c2, l0e)


def _layer_body(hin_ref, agg2_ref, batch_ref, struc_ref,
                w1_ref, c1_ref, w2_ref, c2_ref, lh_ref, lr_ref, bl_ref,
                outc_ref, hin_out_ref):
    pi = pl.program_id(0)
    u = hin_ref[...] + agg2_ref[0] + agg2_ref[1]
    t = jnp.maximum(jnp.dot(u, w1_ref[...], preferred_element_type=jnp.float32)
                    + c1_ref[...], 0.0)
    h = jnp.maximum(jnp.dot(t, w2_ref[...], preferred_element_type=jnp.float32)
                    + c2_ref[...], 0.0)
    b_col = batch_ref[...]
    h = jnp.where(b_col >= 0, h, 0.0)
    ids_c = lax.broadcasted_iota(jnp.int32, (N_GRAPHS, NB), 0)
    PT = jnp.where(b_col.reshape(1, NB) == ids_c, 1.0, 0.0).astype(jnp.float32)
    pooled = jnp.dot(PT, h, preferred_element_type=jnp.float32)

    @pl.when(pi == 0)
    def _():
        outc_ref[...] = (jnp.dot(struc_ref[...], lr_ref[...],
                                 preferred_element_type=jnp.float32)
                         + bl_ref[...])

    outc_ref[...] += jnp.dot(pooled, lh_ref[...],
                             preferred_element_type=jnp.float32)
    hin_out_ref[...] = jnp.zeros((NB, HPAD), jnp.float32)
    hin_out_ref[:, 0:HID] = h
    hin_out_ref[:, HID:HID + INFO_DIM] = hin_ref[:, HID:HID + INFO_DIM]


def _layer_call(hin, agg2, batch_col, struc, w1, c1, w2, c2, lh, lr, bl):
    full = lambda shape: pl.BlockSpec(shape, lambda i: (0,) * len(shape))
    return pl.pallas_call(
        _layer_body,
        grid=(GRID,),
        in_specs=[
            pl.BlockSpec((NB, HPAD), lambda i: (i, 0)),
            pl.BlockSpec((NC, NB, HPAD), lambda i: (0, i, 0)),
            pl.BlockSpec((NB, 1), lambda i: (i, 0)),
            full((N_GRAPHS, INFO_DIM)),
            full((HPAD, HID)), full((1, HID)),
            full((HID, HID)), full((1, HID)),
            full((HID, TGT)), full((INFO_DIM, TGT)), full((1, TGT)),
        ],
        out_specs=(pl.BlockSpec((N_GRAPHS, TGT), lambda i: (0, 0)),
                   pl.BlockSpec((NB, HPAD), lambda i: (i, 0))),
        out_shape=(jax.ShapeDtypeStruct((N_GRAPHS, TGT), jnp.float32),
                   jax.ShapeDtypeStruct((R_PAD, HPAD), jnp.float32)),
    )(hin, agg2, batch_col, struc, w1, c1, w2, c2, lh, lr, bl)


def _sc_agg_body(hin_hbm, src_hbm, dst_hbm, zeros_hbm, out_hbm,
                 src_vm, dst_vm, rows_vm, spmem, sem):
    c = lax.axis_index("c")
    s = lax.axis_index("s")
    wid = c * NS + s
    pltpu.sync_copy(zeros_hbm.at[pl.ds(s * ROWS_PER_TILE, ROWS_PER_TILE)],
                    spmem.at[pl.ds(s * ROWS_PER_TILE, ROWS_PER_TILE)])
    pltpu.sync_copy(src_hbm.at[wid], src_vm)
    pltpu.sync_copy(dst_hbm.at[wid], dst_vm)
    plsc.subcore_barrier()

    pltpu.async_copy(hin_hbm.at[src_vm.at[0]], rows_vm.at[0], sem.at[0])

    def body(j, carry):
        b = lax.rem(j, 2)
        nb = 1 - b

        @pl.when(j + 1 < CHUNKS)
        def _():
            pltpu.async_copy(hin_hbm.at[src_vm.at[j + 1]], rows_vm.at[nb],
                             sem.at[nb])

        pltpu.make_async_copy(hin_hbm.at[src_vm.at[j]], rows_vm.at[b],
                              sem.at[b]).wait()
        pltpu.sync_copy(rows_vm.at[b], spmem.at[dst_vm.at[j]], add=True)
        return carry

    lax.fori_loop(0, CHUNKS, body, 0)
    plsc.subcore_barrier()
    pltpu.sync_copy(spmem.at[pl.ds(s * ROWS_PER_TILE, ROWS_PER_TILE)],
                    out_hbm.at[c, pl.ds(s * ROWS_PER_TILE, ROWS_PER_TILE)])


def _sc_agg(hin, src3d, dst3d, zeros):
    mesh = plsc.VectorSubcoreMesh(core_axis_name="c", subcore_axis_name="s")
    return pl.kernel(
        _sc_agg_body,
        out_type=jax.ShapeDtypeStruct((NC, R_PAD, HPAD), jnp.float32),
        mesh=mesh,
        compiler_params=pltpu.CompilerParams(use_tc_tiling_on_sc=False),
        scratch_types=[
            pltpu.VMEM((CHUNKS, CHUNK), jnp.int32),
            pltpu.VMEM((CHUNKS, CHUNK), jnp.int32),
            pltpu.VMEM((2, CHUNK, HPAD), jnp.float32),
            pltpu.VMEM_SHARED((R_PAD, HPAD), jnp.float32),
            pltpu.SemaphoreType.DMA((2,)),
        ],
    )(hin, src3d, dst3d, zeros)


def kernel(x, edge_index, batch, struc, params):
    p0 = params['mlp0']
    W1f, c1 = _fold_bn(p0['W1'], p0['b1'], p0['bn1'])
    W2f, c2 = _fold_bn(p0['W2'], p0['b2'], p0['bn2'])
    w0 = (W1f[:D_FEAT], W1f[D_FEAT:], c1.reshape(1, HID),
          W2f, c2.reshape(1, HID))
    wl = {}
    for l in range(1, 5):
        pl_ = params['mlp%d' % l]
        W1f, c1 = _fold_bn(pl_['W1'], pl_['b1'], pl_['bn1'])
        W2f, c2 = _fold_bn(pl_['W2'], pl_['b2'], pl_['bn2'])
        W1e = jnp.zeros((HPAD, HID), jnp.float32).at[:HID + INFO_DIM].set(W1f)
        wl[l] = (W1e, c1.reshape(1, HID), W2f, c2.reshape(1, HID))
    lin = {}
    for l in range(5):
        lp = params['lin%d' % l]
        lin[l] = (lp['W'][:HID], lp['W'][HID:], lp['b'].reshape(1, TGT))

    x_pad = jnp.concatenate(
        [x, jnp.zeros((R_PAD - N_NODES, D_FEAT), jnp.float32)])
    batch_col = jnp.concatenate(
        [batch, jnp.full((R_PAD - N_NODES,), -1, jnp.int32)]).reshape(R_PAD, 1)

    src = edge_index[0]
    dst = edge_index[1]
    pad = E_PAD - N_EDGES
    src3d = jnp.concatenate(
        [src, jnp.full((pad,), PAD_ROW, jnp.int32)]).reshape(NW, CHUNKS, CHUNK)
    dst3d = jnp.concatenate(
        [dst, jnp.full((pad,), PAD_ROW, jnp.int32)]).reshape(NW, CHUNKS, CHUNK)
    zeros = jnp.zeros((R_PAD, HPAD), jnp.float32)

    rep = _rep_call(batch_col, struc)
    w1x, w1r, c1, w2, c2 = w0
    l0h, l0r, b0 = lin[0]
    l0e = jnp.concatenate(
        [l0h, l0r, b0, jnp.zeros((HPAD - HID - INFO_DIM - 1, TGT), jnp.float32)])
    out, hin = _layer0_call(x_pad, rep, batch_col, w1x, w1r, c1, w2, c2, l0e)

    for l in range(1, 5):
        agg2 = _sc_agg(hin, src3d, dst3d, zeros)
        w1, c1, w2, c2 = wl[l]
        lh, lr, bl = lin[l]
        outc, hin = _layer_call(hin, agg2, batch_col, struc,
                                w1, c1, w2, c2, lh, lr, bl)
        out = out + outc
    return out


# speedup vs baseline: 5.4876x; 1.5322x over previous
import jax
import jax.numpy as jnp
from jax import lax
from jax.experimental import pallas as pl
from jax.experimental.pallas import tpu as pltpu
from jax.experimental.pallas import tpu_sc as plsc

N_NODES = 10000
D_FEAT = 128
INFO_DIM = 8
HID = 64
TGT = 10
N_GRAPHS = 128
EPS_BN = 1e-5

R_PAD = 10080
NB = 1008
GRID = R_PAD // NB
PAD_ROW = R_PAD - 1
HPAD = 80

NC = 2
NS = 16
NW = NC * NS
ROWS_PER_TILE = R_PAD // NS

CHUNK = 128
N_EDGES = 320000
CHUNKS = -(-N_EDGES // (NW * CHUNK))
E_PAD = NW * CHUNKS * CHUNK


def _fold_bn(Wp, bp, bn):
    s = bn['g'] / jnp.sqrt(bn['rv'] + EPS_BN)
    Wf = Wp * s[None, :]
    cf = (bp - bn['rm']) * s + bn['b']
    return Wf, cf


def _rep_body(batch_ref, struc_ref, rep_ref):
    b_col = batch_ref[...]
    ids_r = lax.broadcasted_iota(jnp.int32, (NB, N_GRAPHS), 1)
    P = jnp.where(b_col == ids_r, 1.0, 0.0).astype(jnp.float32)
    rep_ref[...] = jnp.dot(P, struc_ref[...], preferred_element_type=jnp.float32)


def _rep_call(batch_col, struc):
    return pl.pallas_call(
        _rep_body,
        grid=(GRID,),
        in_specs=[pl.BlockSpec((NB, 1), lambda i: (i, 0)),
                  pl.BlockSpec((N_GRAPHS, INFO_DIM), lambda i: (0, 0))],
        out_specs=pl.BlockSpec((NB, INFO_DIM), lambda i: (i, 0)),
        out_shape=jax.ShapeDtypeStruct((R_PAD, INFO_DIM), jnp.float32),
    )(batch_col, struc)


def _layer0_body(x_ref, rep_ref, batch_ref, brow_ref, w1x_ref, w1r_ref, c1_ref,
                 w2_ref, c2_ref, l0e_ref, out0_ref, hin_ref):
    pi = pl.program_id(0)
    x = x_ref[...]
    b_col = batch_ref[...]
    b_row = brow_ref[0]
    rep = rep_ref[...]
    t = jnp.dot(x, w1x_ref[...], preferred_element_type=jnp.float32)
    t += jnp.dot(rep, w1r_ref[...], preferred_element_type=jnp.float32)
    t = jnp.maximum(t + c1_ref[...], 0.0)
    h = jnp.maximum(jnp.dot(t, w2_ref[...], preferred_element_type=jnp.float32)
                    + c2_ref[...], 0.0)
    h = jnp.where(b_col >= 0, h, 0.0)
    mask = jnp.where(b_col >= 0, 1.0, 0.0).astype(jnp.float32)
    v = jnp.concatenate(
        [h, rep, mask, jnp.zeros((NB, HPAD - HID - INFO_DIM - 1), jnp.float32)],
        axis=1)
    ids_c = lax.broadcasted_iota(jnp.int32, (N_GRAPHS, NB), 0)
    PT = jnp.where(b_row == ids_c, 1.0, 0.0).astype(jnp.float32)
    pooled = jnp.dot(PT, v, preferred_element_type=jnp.float32)

    @pl.when(pi == 0)
    def _():
        out0_ref[...] = jnp.zeros((N_GRAPHS, TGT), jnp.float32)

    out0_ref[...] += jnp.dot(pooled, l0e_ref[...],
                             preferred_element_type=jnp.float32)
    hin_ref[...] = v


def _layer0_call(x_pad, rep, batch_col, batch_row, w1x, w1r, c1, w2, c2, l0e):
    full = lambda shape: pl.BlockSpec(shape, lambda i: (0,) * len(shape))
    return pl.pallas_call(
        _layer0_body,
        grid=(GRID,),
        in_specs=[
            pl.BlockSpec((NB, D_FEAT), lambda i: (i, 0)),
            pl.BlockSpec((NB, INFO_DIM), lambda i: (i, 0)),
            pl.BlockSpec((NB, 1), lambda i: (i, 0)),
            pl.BlockSpec((1, 1, NB), lambda i: (i, 0, 0)),
            full((D_FEAT, HID)), full((INFO_DIM, HID)), full((1, HID)),
            full((HID, HID)), full((1, HID)),
            full((HPAD, TGT)),
        ],
        out_specs=(pl.BlockSpec((N_GRAPHS, TGT), lambda i: (0, 0)),
                   pl.BlockSpec((NB, HPAD), lambda i: (i, 0))),
        out_shape=(jax.ShapeDtypeStruct((N_GRAPHS, TGT), jnp.float32),
                   jax.ShapeDtypeStruct((R_PAD, HPAD), jnp.float32)),
    )(x_pad, rep, batch_col, batch_row, w1x, w1r, c1, w2, c2, l0e)


def _layer_body(hin_ref, agg2_ref, batch_ref, brow_ref, struc_ref,
                w1_ref, c1_ref, w2_ref, c2_ref, lh_ref, lr_ref, bl_ref,
                outc_ref, hin_out_ref):
    pi = pl.program_id(0)
    u = hin_ref[...] + agg2_ref[0] + agg2_ref[1]
    t = jnp.maximum(jnp.dot(u, w1_ref[...], preferred_element_type=jnp.float32)
                    + c1_ref[...], 0.0)
    h = jnp.maximum(jnp.dot(t, w2_ref[...], preferred_element_type=jnp.float32)
                    + c2_ref[...], 0.0)
    b_col = batch_ref[...]
    h = jnp.where(b_col >= 0, h, 0.0)
    ids_c = lax.broadcasted_iota(jnp.int32, (N_GRAPHS, NB), 0)
    PT = jnp.where(brow_ref[0] == ids_c, 1.0, 0.0).astype(jnp.float32)
    pooled = jnp.dot(PT, h, preferred_element_type=jnp.float32)

    @pl.when(pi == 0)
    def _():
        outc_ref[...] = (jnp.dot(struc_ref[...], lr_ref[...],
                                 preferred_element_type=jnp.float32)
                         + bl_ref[...])

    outc_ref[...] += jnp.dot(pooled, lh_ref[...],
                             preferred_element_type=jnp.float32)
    hin_out_ref[...] = jnp.zeros((NB, HPAD), jnp.float32)
    hin_out_ref[:, 0:HID] = h
    hin_out_ref[:, HID:HID + INFO_DIM] = hin_ref[:, HID:HID + INFO_DIM]


def _layer_call(hin, agg2, batch_col, batch_row, struc, w1, c1, w2, c2, lh, lr, bl):
    full = lambda shape: pl.BlockSpec(shape, lambda i: (0,) * len(shape))
    return pl.pallas_call(
        _layer_body,
        grid=(GRID,),
        in_specs=[
            pl.BlockSpec((NB, HPAD), lambda i: (i, 0)),
            pl.BlockSpec((NC, NB, HPAD), lambda i: (0, i, 0)),
            pl.BlockSpec((NB, 1), lambda i: (i, 0)),
            pl.BlockSpec((1, 1, NB), lambda i: (i, 0, 0)),
            full((N_GRAPHS, INFO_DIM)),
            full((HPAD, HID)), full((1, HID)),
            full((HID, HID)), full((1, HID)),
            full((HID, TGT)), full((INFO_DIM, TGT)), full((1, TGT)),
        ],
        out_specs=(pl.BlockSpec((N_GRAPHS, TGT), lambda i: (0, 0)),
                   pl.BlockSpec((NB, HPAD), lambda i: (i, 0))),
        out_shape=(jax.ShapeDtypeStruct((N_GRAPHS, TGT), jnp.float32),
                   jax.ShapeDtypeStruct((R_PAD, HPAD), jnp.float32)),
    )(hin, agg2, batch_col, batch_row, struc, w1, c1, w2, c2, lh, lr, bl)


def _sc_agg_body(hin_hbm, src_hbm, dst_hbm, zeros_hbm, out_hbm,
                 src_vm, dst_vm, rows_vm, spmem, sem):
    c = lax.axis_index("c")
    s = lax.axis_index("s")
    wid = c * NS + s
    pltpu.sync_copy(zeros_hbm.at[pl.ds(s * ROWS_PER_TILE, ROWS_PER_TILE)],
                    spmem.at[pl.ds(s * ROWS_PER_TILE, ROWS_PER_TILE)])
    pltpu.sync_copy(src_hbm.at[wid], src_vm)
    pltpu.sync_copy(dst_hbm.at[wid], dst_vm)
    plsc.subcore_barrier()

    pltpu.async_copy(hin_hbm.at[src_vm.at[0]], rows_vm.at[0], sem.at[0])

    def body(j, carry):
        b = lax.rem(j, 2)
        nb = 1 - b

        @pl.when(j + 1 < CHUNKS)
        def _():
            pltpu.async_copy(hin_hbm.at[src_vm.at[j + 1]], rows_vm.at[nb],
                             sem.at[nb])

        pltpu.make_async_copy(hin_hbm.at[src_vm.at[j]], rows_vm.at[b],
                              sem.at[b]).wait()
        pltpu.sync_copy(rows_vm.at[b], spmem.at[dst_vm.at[j]], add=True)
        return carry

    lax.fori_loop(0, CHUNKS, body, 0)
    plsc.subcore_barrier()
    pltpu.sync_copy(spmem.at[pl.ds(s * ROWS_PER_TILE, ROWS_PER_TILE)],
                    out_hbm.at[c, pl.ds(s * ROWS_PER_TILE, ROWS_PER_TILE)])


def _sc_agg(hin, src3d, dst3d, zeros):
    mesh = plsc.VectorSubcoreMesh(core_axis_name="c", subcore_axis_name="s")
    return pl.kernel(
        _sc_agg_body,
        out_type=jax.ShapeDtypeStruct((NC, R_PAD, HPAD), jnp.float32),
        mesh=mesh,
        compiler_params=pltpu.CompilerParams(use_tc_tiling_on_sc=False),
        scratch_types=[
            pltpu.VMEM((CHUNKS, CHUNK), jnp.int32),
            pltpu.VMEM((CHUNKS, CHUNK), jnp.int32),
            pltpu.VMEM((2, CHUNK, HPAD), jnp.float32),
            pltpu.VMEM_SHARED((R_PAD, HPAD), jnp.float32),
            pltpu.SemaphoreType.DMA((2,)),
        ],
    )(hin, src3d, dst3d, zeros)


def kernel(x, edge_index, batch, struc, params):
    p0 = params['mlp0']
    W1f, c1 = _fold_bn(p0['W1'], p0['b1'], p0['bn1'])
    W2f, c2 = _fold_bn(p0['W2'], p0['b2'], p0['bn2'])
    w0 = (W1f[:D_FEAT], W1f[D_FEAT:], c1.reshape(1, HID),
          W2f, c2.reshape(1, HID))
    wl = {}
    for l in range(1, 5):
        pl_ = params['mlp%d' % l]
        W1f, c1 = _fold_bn(pl_['W1'], pl_['b1'], pl_['bn1'])
        W2f, c2 = _fold_bn(pl_['W2'], pl_['b2'], pl_['bn2'])
        W1e = jnp.zeros((HPAD, HID), jnp.float32).at[:HID + INFO_DIM].set(W1f)
        wl[l] = (W1e, c1.reshape(1, HID), W2f, c2.reshape(1, HID))
    lin = {}
    for l in range(5):
        lp = params['lin%d' % l]
        lin[l] = (lp['W'][:HID], lp['W'][HID:], lp['b'].reshape(1, TGT))

    x_pad = jnp.concatenate(
        [x, jnp.zeros((R_PAD - N_NODES, D_FEAT), jnp.float32)])
    batch_pad = jnp.concatenate(
        [batch, jnp.full((R_PAD - N_NODES,), -1, jnp.int32)])
    batch_col = batch_pad.reshape(R_PAD, 1)
    batch_row = batch_pad.reshape(GRID, 1, NB)

    src = edge_index[0]
    dst = edge_index[1]
    pad = E_PAD - N_EDGES
    src3d = jnp.concatenate(
        [src, jnp.full((pad,), PAD_ROW, jnp.int32)]).reshape(NW, CHUNKS, CHUNK)
    dst3d = jnp.concatenate(
        [dst, jnp.arange(pad, dtype=jnp.int32) % N_NODES]).reshape(NW, CHUNKS, CHUNK)
    zeros = jnp.zeros((R_PAD, HPAD), jnp.float32)

    rep = _rep_call(batch_col, struc)
    w1x, w1r, c1, w2, c2 = w0
    l0h, l0r, b0 = lin[0]
    l0e = jnp.concatenate(
        [l0h, l0r, b0, jnp.zeros((HPAD - HID - INFO_DIM - 1, TGT), jnp.float32)])
    out, hin = _layer0_call(x_pad, rep, batch_col, batch_row,
                            w1x, w1r, c1, w2, c2, l0e)

    for l in range(1, 5):
        agg2 = _sc_agg(hin, src3d, dst3d, zeros)
        w1, c1, w2, c2 = wl[l]
        lh, lr, bl = lin[l]
        outc, hin = _layer_call(hin, agg2, batch_col, batch_row, struc,
                                w1, c1, w2, c2, lh, lr, bl)
        out = out + outc
    return out


# speedup vs baseline: 10.2549x; 1.8687x over previous
import jax
import jax.numpy as jnp
from jax import lax
from jax.experimental import pallas as pl
from jax.experimental.pallas import tpu as pltpu
from jax.experimental.pallas import tpu_sc as plsc

N_NODES = 10000
D_FEAT = 128
INFO_DIM = 8
HID = 64
TGT = 10
N_GRAPHS = 128
EPS_BN = 1e-5

R_PAD = 10080
NB = 1008
GRID = R_PAD // NB
PAD_ROW = R_PAD - 1
HPAD = 80

NC = 2
NS = 16
NW = NC * NS
ROWS_PER_TILE = R_PAD // NS

CHUNK = 128
N_EDGES = 320000
CHUNKS = -(-N_EDGES // (NS * CHUNK))
E_PAD = NS * CHUNKS * CHUNK
HHALF = HPAD // 2


def _fold_bn(Wp, bp, bn):
    s = bn['g'] / jnp.sqrt(bn['rv'] + EPS_BN)
    Wf = Wp * s[None, :]
    cf = (bp - bn['rm']) * s + bn['b']
    return Wf, cf


def _rep_body(batch_ref, struc_ref, rep_ref):
    b_col = batch_ref[...]
    ids_r = lax.broadcasted_iota(jnp.int32, (NB, N_GRAPHS), 1)
    P = jnp.where(b_col == ids_r, 1.0, 0.0).astype(jnp.float32)
    rep_ref[...] = jnp.dot(P, struc_ref[...], preferred_element_type=jnp.float32)


def _rep_call(batch_col, struc):
    return pl.pallas_call(
        _rep_body,
        grid=(GRID,),
        in_specs=[pl.BlockSpec((NB, 1), lambda i: (i, 0)),
                  pl.BlockSpec((N_GRAPHS, INFO_DIM), lambda i: (0, 0))],
        out_specs=pl.BlockSpec((NB, INFO_DIM), lambda i: (i, 0)),
        out_shape=jax.ShapeDtypeStruct((R_PAD, INFO_DIM), jnp.float32),
    )(batch_col, struc)


def _layer0_body(x_ref, rep_ref, batch_ref, brow_ref, w1x_ref, w1r_ref, c1_ref,
                 w2_ref, c2_ref, l0e_ref, out0_ref, hin_ref):
    pi = pl.program_id(0)
    x = x_ref[...]
    b_col = batch_ref[...]
    b_row = brow_ref[0]
    rep = rep_ref[...]
    t = jnp.dot(x, w1x_ref[...], preferred_element_type=jnp.float32)
    t += jnp.dot(rep, w1r_ref[...], preferred_element_type=jnp.float32)
    t = jnp.maximum(t + c1_ref[...], 0.0)
    h = jnp.maximum(jnp.dot(t, w2_ref[...], preferred_element_type=jnp.float32)
                    + c2_ref[...], 0.0)
    h = jnp.where(b_col >= 0, h, 0.0)
    mask = jnp.where(b_col >= 0, 1.0, 0.0).astype(jnp.float32)
    v = jnp.concatenate(
        [h, rep, mask, jnp.zeros((NB, HPAD - HID - INFO_DIM - 1), jnp.float32)],
        axis=1)
    ids_c = lax.broadcasted_iota(jnp.int32, (N_GRAPHS, NB), 0)
    PT = jnp.where(b_row == ids_c, 1.0, 0.0).astype(jnp.float32)
    pooled = jnp.dot(PT, v, preferred_element_type=jnp.float32)

    @pl.when(pi == 0)
    def _():
        out0_ref[...] = jnp.zeros((N_GRAPHS, TGT), jnp.float32)

    out0_ref[...] += jnp.dot(pooled, l0e_ref[...],
                             preferred_element_type=jnp.float32)
    hin_ref[0] = v[:, 0:HHALF]
    hin_ref[1] = v[:, HHALF:HPAD]


def _layer0_call(x_pad, rep, batch_col, batch_row, w1x, w1r, c1, w2, c2, l0e):
    full = lambda shape: pl.BlockSpec(shape, lambda i: (0,) * len(shape))
    return pl.pallas_call(
        _layer0_body,
        grid=(GRID,),
        in_specs=[
            pl.BlockSpec((NB, D_FEAT), lambda i: (i, 0)),
            pl.BlockSpec((NB, INFO_DIM), lambda i: (i, 0)),
            pl.BlockSpec((NB, 1), lambda i: (i, 0)),
            pl.BlockSpec((1, 1, NB), lambda i: (i, 0, 0)),
            full((D_FEAT, HID)), full((INFO_DIM, HID)), full((1, HID)),
            full((HID, HID)), full((1, HID)),
            full((HPAD, TGT)),
        ],
        out_specs=(pl.BlockSpec((N_GRAPHS, TGT), lambda i: (0, 0)),
                   pl.BlockSpec((NC, NB, HHALF), lambda i: (0, i, 0))),
        out_shape=(jax.ShapeDtypeStruct((N_GRAPHS, TGT), jnp.float32),
                   jax.ShapeDtypeStruct((NC, R_PAD, HHALF), jnp.float32)),
    )(x_pad, rep, batch_col, batch_row, w1x, w1r, c1, w2, c2, l0e)


def _layer_body(hin_ref, agg2_ref, batch_ref, brow_ref, struc_ref,
                w1_ref, c1_ref, w2_ref, c2_ref, lh_ref, lr_ref, bl_ref,
                outc_ref, hin_out_ref):
    pi = pl.program_id(0)
    u = jnp.concatenate([hin_ref[0] + agg2_ref[0],
                         hin_ref[1] + agg2_ref[1]], axis=1)
    t = jnp.maximum(jnp.dot(u, w1_ref[...], preferred_element_type=jnp.float32)
                    + c1_ref[...], 0.0)
    h = jnp.maximum(jnp.dot(t, w2_ref[...], preferred_element_type=jnp.float32)
                    + c2_ref[...], 0.0)
    b_col = batch_ref[...]
    h = jnp.where(b_col >= 0, h, 0.0)
    ids_c = lax.broadcasted_iota(jnp.int32, (N_GRAPHS, NB), 0)
    PT = jnp.where(brow_ref[0] == ids_c, 1.0, 0.0).astype(jnp.float32)
    pooled = jnp.dot(PT, h, preferred_element_type=jnp.float32)

    @pl.when(pi == 0)
    def _():
        outc_ref[...] = (jnp.dot(struc_ref[...], lr_ref[...],
                                 preferred_element_type=jnp.float32)
                         + bl_ref[...])

    outc_ref[...] += jnp.dot(pooled, lh_ref[...],
                             preferred_element_type=jnp.float32)
    hin_out_ref[0] = h[:, 0:HHALF]
    hin_out_ref[1] = jnp.concatenate(
        [h[:, HHALF:HID], hin_ref[1][:, HID - HHALF:HID - HHALF + INFO_DIM],
         jnp.zeros((NB, HPAD - HID - INFO_DIM), jnp.float32)], axis=1)


def _layer_call(hin, agg2, batch_col, batch_row, struc, w1, c1, w2, c2, lh, lr, bl):
    full = lambda shape: pl.BlockSpec(shape, lambda i: (0,) * len(shape))
    return pl.pallas_call(
        _layer_body,
        grid=(GRID,),
        in_specs=[
            pl.BlockSpec((NC, NB, HHALF), lambda i: (0, i, 0)),
            pl.BlockSpec((NC, NB, HHALF), lambda i: (0, i, 0)),
            pl.BlockSpec((NB, 1), lambda i: (i, 0)),
            pl.BlockSpec((1, 1, NB), lambda i: (i, 0, 0)),
            full((N_GRAPHS, INFO_DIM)),
            full((HPAD, HID)), full((1, HID)),
            full((HID, HID)), full((1, HID)),
            full((HID, TGT)), full((INFO_DIM, TGT)), full((1, TGT)),
        ],
        out_specs=(pl.BlockSpec((N_GRAPHS, TGT), lambda i: (0, 0)),
                   pl.BlockSpec((NC, NB, HHALF), lambda i: (0, i, 0))),
        out_shape=(jax.ShapeDtypeStruct((N_GRAPHS, TGT), jnp.float32),
                   jax.ShapeDtypeStruct((NC, R_PAD, HHALF), jnp.float32)),
    )(hin, agg2, batch_col, batch_row, struc, w1, c1, w2, c2, lh, lr, bl)


def _sc_agg_body(hin_hbm, src_hbm, dst_hbm, zeros_hbm, out_hbm,
                 src_vm, dst_vm, rows_vm, spmem, hin_sp, sem):
    c = lax.axis_index("c")
    s = lax.axis_index("s")
    pltpu.sync_copy(zeros_hbm.at[pl.ds(s * ROWS_PER_TILE, ROWS_PER_TILE)],
                    spmem.at[pl.ds(s * ROWS_PER_TILE, ROWS_PER_TILE)])
    pltpu.sync_copy(hin_hbm.at[c, pl.ds(s * ROWS_PER_TILE, ROWS_PER_TILE)],
                    hin_sp.at[pl.ds(s * ROWS_PER_TILE, ROWS_PER_TILE)])
    pltpu.sync_copy(src_hbm.at[s], src_vm)
    pltpu.sync_copy(dst_hbm.at[s], dst_vm)
    plsc.subcore_barrier()

    pltpu.async_copy(hin_sp.at[src_vm.at[0]], rows_vm.at[0], sem.at[0])

    def body(j, carry):
        b = lax.rem(j, 2)
        nb = 1 - b

        @pl.when(j + 1 < CHUNKS)
        def _():
            pltpu.async_copy(hin_sp.at[src_vm.at[j + 1]], rows_vm.at[nb],
                             sem.at[nb])

        pltpu.make_async_copy(hin_sp.at[src_vm.at[j]], rows_vm.at[b],
                              sem.at[b]).wait()
        pltpu.sync_copy(rows_vm.at[b], spmem.at[dst_vm.at[j]], add=True)
        return carry

    lax.fori_loop(0, CHUNKS, body, 0)
    plsc.subcore_barrier()
    pltpu.sync_copy(spmem.at[pl.ds(s * ROWS_PER_TILE, ROWS_PER_TILE)],
                    out_hbm.at[c, pl.ds(s * ROWS_PER_TILE, ROWS_PER_TILE)])


def _sc_agg(hin2, src3d, dst3d, zeros):
    mesh = plsc.VectorSubcoreMesh(core_axis_name="c", subcore_axis_name="s")
    return pl.kernel(
        _sc_agg_body,
        out_type=jax.ShapeDtypeStruct((NC, R_PAD, HHALF), jnp.float32),
        mesh=mesh,
        compiler_params=pltpu.CompilerParams(use_tc_tiling_on_sc=False),
        scratch_types=[
            pltpu.VMEM((CHUNKS, CHUNK), jnp.int32),
            pltpu.VMEM((CHUNKS, CHUNK), jnp.int32),
            pltpu.VMEM((2, CHUNK, HHALF), jnp.float32),
            pltpu.VMEM_SHARED((R_PAD, HHALF), jnp.float32),
            pltpu.VMEM_SHARED((R_PAD, HHALF), jnp.float32),
            pltpu.SemaphoreType.DMA((2,)),
        ],
    )(hin2, src3d, dst3d, zeros)


def kernel(x, edge_index, batch, struc, params):
    p0 = params['mlp0']
    W1f, c1 = _fold_bn(p0['W1'], p0['b1'], p0['bn1'])
    W2f, c2 = _fold_bn(p0['W2'], p0['b2'], p0['bn2'])
    w0 = (W1f[:D_FEAT], W1f[D_FEAT:], c1.reshape(1, HID),
          W2f, c2.reshape(1, HID))
    wl = {}
    for l in range(1, 5):
        pl_ = params['mlp%d' % l]
        W1f, c1 = _fold_bn(pl_['W1'], pl_['b1'], pl_['bn1'])
        W2f, c2 = _fold_bn(pl_['W2'], pl_['b2'], pl_['bn2'])
        W1e = jnp.zeros((HPAD, HID), jnp.float32).at[:HID + INFO_DIM].set(W1f)
        wl[l] = (W1e, c1.reshape(1, HID), W2f, c2.reshape(1, HID))
    lin = {}
    for l in range(5):
        lp = params['lin%d' % l]
        lin[l] = (lp['W'][:HID], lp['W'][HID:], lp['b'].reshape(1, TGT))

    x_pad = jnp.concatenate(
        [x, jnp.zeros((R_PAD - N_NODES, D_FEAT), jnp.float32)])
    batch_pad = jnp.concatenate(
        [batch, jnp.full((R_PAD - N_NODES,), -1, jnp.int32)])
    batch_col = batch_pad.reshape(R_PAD, 1)
    batch_row = batch_pad.reshape(GRID, 1, NB)

    src = edge_index[0]
    dst = edge_index[1]
    pad = E_PAD - N_EDGES
    src3d = jnp.concatenate(
        [src, jnp.full((pad,), PAD_ROW, jnp.int32)]).reshape(NS, CHUNKS, CHUNK)
    dst3d = jnp.concatenate(
        [dst, jnp.arange(pad, dtype=jnp.int32) % N_NODES]).reshape(NS, CHUNKS, CHUNK)
    zeros = jnp.zeros((R_PAD, HHALF), jnp.float32)

    rep = _rep_call(batch_col, struc)
    w1x, w1r, c1, w2, c2 = w0
    l0h, l0r, b0 = lin[0]
    l0e = jnp.concatenate(
        [l0h, l0r, b0, jnp.zeros((HPAD - HID - INFO_DIM - 1, TGT), jnp.float32)])
    out, hin = _layer0_call(x_pad, rep, batch_col, batch_row,
                            w1x, w1r, c1, w2, c2, l0e)

    for l in range(1, 5):
        agg2 = _sc_agg(hin, src3d, dst3d, zeros)
        w1, c1, w2, c2 = wl[l]
        lh, lr, bl = lin[l]
        outc, hin = _layer_call(hin, agg2, batch_col, batch_row, struc,
                                w1, c1, w2, c2, lh, lr, bl)
        out = out + outc
    return out


# speedup vs baseline: 11.4056x; 1.1122x over previous
import jax
import jax.numpy as jnp
from jax import lax
from jax.experimental import pallas as pl
from jax.experimental.pallas import tpu as pltpu
from jax.experimental.pallas import tpu_sc as plsc

N_NODES = 10000
D_FEAT = 128
INFO_DIM = 8
HID = 64
TGT = 10
N_GRAPHS = 128
EPS_BN = 1e-5

R_PAD = 10080
NB = 1008
GRID = R_PAD // NB
PAD_ROW = R_PAD - 1
HPAD = 80

NC = 2
NS = 16
NW = NC * NS
ROWS_PER_TILE = R_PAD // NS

CHUNK = 128
N_EDGES = 320000
CHUNKS = -(-N_EDGES // (NS * CHUNK))
E_PAD = NS * CHUNKS * CHUNK
HHALF = HPAD // 2


def _fold_bn(Wp, bp, bn):
    s = bn['g'] / jnp.sqrt(bn['rv'] + EPS_BN)
    Wf = Wp * s[None, :]
    cf = (bp - bn['rm']) * s + bn['b']
    return Wf, cf


def _rep_body(batch_ref, struc_ref, rep_ref):
    b_col = batch_ref[...]
    ids_r = lax.broadcasted_iota(jnp.int32, (NB, N_GRAPHS), 1)
    P = jnp.where(b_col == ids_r, 1.0, 0.0).astype(jnp.float32)
    rep_ref[...] = jnp.dot(P, struc_ref[...], preferred_element_type=jnp.float32)


def _rep_call(batch_col, struc):
    return pl.pallas_call(
        _rep_body,
        grid=(GRID,),
        in_specs=[pl.BlockSpec((NB, 1), lambda i: (i, 0)),
                  pl.BlockSpec((N_GRAPHS, INFO_DIM), lambda i: (0, 0))],
        out_specs=pl.BlockSpec((NB, INFO_DIM), lambda i: (i, 0)),
        out_shape=jax.ShapeDtypeStruct((R_PAD, INFO_DIM), jnp.float32),
    )(batch_col, struc)


def _layer0_body(x_ref, rep_ref, batch_ref, brow_ref, w1x_ref, w1r_ref, c1_ref,
                 w2_ref, c2_ref, l0e_ref, out0_ref, hin_ref):
    pi = pl.program_id(0)
    x = x_ref[...]
    b_col = batch_ref[...]
    b_row = brow_ref[0]
    rep = rep_ref[...]
    t = jnp.dot(x, w1x_ref[...], preferred_element_type=jnp.float32)
    t += jnp.dot(rep, w1r_ref[...], preferred_element_type=jnp.float32)
    t = jnp.maximum(t + c1_ref[...], 0.0)
    h = jnp.maximum(jnp.dot(t, w2_ref[...], preferred_element_type=jnp.float32)
                    + c2_ref[...], 0.0)
    h = jnp.where(b_col >= 0, h, 0.0)
    mask = jnp.where(b_col >= 0, 1.0, 0.0).astype(jnp.float32)
    v = jnp.concatenate(
        [h, rep, mask, jnp.zeros((NB, HPAD - HID - INFO_DIM - 1), jnp.float32)],
        axis=1)
    ids_c = lax.broadcasted_iota(jnp.int32, (N_GRAPHS, NB), 0)
    PT = jnp.where(b_row == ids_c, 1.0, 0.0).astype(jnp.float32)
    pooled = jnp.dot(PT, v, preferred_element_type=jnp.float32)

    @pl.when(pi == 0)
    def _():
        out0_ref[...] = jnp.zeros((N_GRAPHS, TGT), jnp.float32)

    out0_ref[...] += jnp.dot(pooled, l0e_ref[...],
                             preferred_element_type=jnp.float32)
    hin_ref[0] = v[:, 0:HHALF]
    hin_ref[1] = v[:, HHALF:HPAD]


def _layer0_call(x_pad, rep, batch_col, batch_row, w1x, w1r, c1, w2, c2, l0e):
    full = lambda shape: pl.BlockSpec(shape, lambda i: (0,) * len(shape))
    return pl.pallas_call(
        _layer0_body,
        grid=(GRID,),
        in_specs=[
            pl.BlockSpec((NB, D_FEAT), lambda i: (i, 0)),
            pl.BlockSpec((NB, INFO_DIM), lambda i: (i, 0)),
            pl.BlockSpec((NB, 1), lambda i: (i, 0)),
            pl.BlockSpec((1, 1, NB), lambda i: (i, 0, 0)),
            full((D_FEAT, HID)), full((INFO_DIM, HID)), full((1, HID)),
            full((HID, HID)), full((1, HID)),
            full((HPAD, TGT)),
        ],
        out_specs=(pl.BlockSpec((N_GRAPHS, TGT), lambda i: (0, 0)),
                   pl.BlockSpec((NC, NB, HHALF), lambda i: (0, i, 0))),
        out_shape=(jax.ShapeDtypeStruct((N_GRAPHS, TGT), jnp.float32),
                   jax.ShapeDtypeStruct((NC, R_PAD, HHALF), jnp.float32)),
    )(x_pad, rep, batch_col, batch_row, w1x, w1r, c1, w2, c2, l0e)


def _layer_body(hin_ref, agg2_ref, batch_ref, brow_ref, struc_ref,
                w1_ref, c1_ref, w2_ref, c2_ref, lh_ref, lr_ref, bl_ref,
                outc_ref, hin_out_ref):
    pi = pl.program_id(0)
    u = jnp.concatenate([hin_ref[0] + agg2_ref[0],
                         hin_ref[1] + agg2_ref[1]], axis=1)
    t = jnp.maximum(jnp.dot(u, w1_ref[...], preferred_element_type=jnp.float32)
                    + c1_ref[...], 0.0)
    h = jnp.maximum(jnp.dot(t, w2_ref[...], preferred_element_type=jnp.float32)
                    + c2_ref[...], 0.0)
    b_col = batch_ref[...]
    h = jnp.where(b_col >= 0, h, 0.0)
    ids_c = lax.broadcasted_iota(jnp.int32, (N_GRAPHS, NB), 0)
    PT = jnp.where(brow_ref[0] == ids_c, 1.0, 0.0).astype(jnp.float32)
    pooled = jnp.dot(PT, h, preferred_element_type=jnp.float32)

    @pl.when(pi == 0)
    def _():
        outc_ref[...] = (jnp.dot(struc_ref[...], lr_ref[...],
                                 preferred_element_type=jnp.float32)
                         + bl_ref[...])

    outc_ref[...] += jnp.dot(pooled, lh_ref[...],
                             preferred_element_type=jnp.float32)
    hin_out_ref[0] = h[:, 0:HHALF]
    hin_out_ref[1] = jnp.concatenate(
        [h[:, HHALF:HID], hin_ref[1][:, HID - HHALF:HID - HHALF + INFO_DIM],
         jnp.zeros((NB, HPAD - HID - INFO_DIM), jnp.float32)], axis=1)


def _layer_call(hin, agg2, batch_col, batch_row, struc, w1, c1, w2, c2, lh, lr, bl):
    full = lambda shape: pl.BlockSpec(shape, lambda i: (0,) * len(shape))
    return pl.pallas_call(
        _layer_body,
        grid=(GRID,),
        in_specs=[
            pl.BlockSpec((NC, NB, HHALF), lambda i: (0, i, 0)),
            pl.BlockSpec((NC, NB, HHALF), lambda i: (0, i, 0)),
            pl.BlockSpec((NB, 1), lambda i: (i, 0)),
            pl.BlockSpec((1, 1, NB), lambda i: (i, 0, 0)),
            full((N_GRAPHS, INFO_DIM)),
            full((HPAD, HID)), full((1, HID)),
            full((HID, HID)), full((1, HID)),
            full((HID, TGT)), full((INFO_DIM, TGT)), full((1, TGT)),
        ],
        out_specs=(pl.BlockSpec((N_GRAPHS, TGT), lambda i: (0, 0)),
                   pl.BlockSpec((NC, NB, HHALF), lambda i: (0, i, 0))),
        out_shape=(jax.ShapeDtypeStruct((N_GRAPHS, TGT), jnp.float32),
                   jax.ShapeDtypeStruct((NC, R_PAD, HHALF), jnp.float32)),
    )(hin, agg2, batch_col, batch_row, struc, w1, c1, w2, c2, lh, lr, bl)


NBUF = 4


def _sc_agg_body(hin_hbm, src_hbm, dst_hbm, zeros_hbm, out_hbm,
                 src_vm, dst_vm, rows_vm, spmem, hin_sp, gsem, ssem):
    c = lax.axis_index("c")
    s = lax.axis_index("s")
    pltpu.sync_copy(zeros_hbm.at[pl.ds(s * ROWS_PER_TILE, ROWS_PER_TILE)],
                    spmem.at[pl.ds(s * ROWS_PER_TILE, ROWS_PER_TILE)])
    pltpu.sync_copy(hin_hbm.at[c, pl.ds(s * ROWS_PER_TILE, ROWS_PER_TILE)],
                    hin_sp.at[pl.ds(s * ROWS_PER_TILE, ROWS_PER_TILE)])
    pltpu.sync_copy(src_hbm.at[s], src_vm)
    pltpu.sync_copy(dst_hbm.at[s], dst_vm)
    plsc.subcore_barrier()

    pltpu.async_copy(hin_sp.at[src_vm.at[0]], rows_vm.at[0], gsem.at[0])

    def body(j, carry):
        @pl.when(j + 1 < CHUNKS)
        def _():
            nb = lax.rem(j + 1, NBUF)

            @pl.when(j >= NBUF - 1)
            def _():
                pltpu.make_async_copy(
                    rows_vm.at[nb], spmem.at[dst_vm.at[j + 1 - NBUF]],
                    ssem.at[nb]).wait()

            pltpu.async_copy(hin_sp.at[src_vm.at[j + 1]], rows_vm.at[nb],
                             gsem.at[nb])

        b = lax.rem(j, NBUF)
        pltpu.make_async_copy(hin_sp.at[src_vm.at[j]], rows_vm.at[b],
                              gsem.at[b]).wait()
        pltpu.async_copy(rows_vm.at[b], spmem.at[dst_vm.at[j]], ssem.at[b],
                         add=True)
        return carry

    lax.fori_loop(0, CHUNKS, body, 0)
    for k in range(max(0, CHUNKS - (NBUF - 1)), CHUNKS):
        pltpu.make_async_copy(rows_vm.at[k % NBUF], spmem.at[dst_vm.at[k]],
                              ssem.at[k % NBUF]).wait()
    plsc.subcore_barrier()
    pltpu.sync_copy(spmem.at[pl.ds(s * ROWS_PER_TILE, ROWS_PER_TILE)],
                    out_hbm.at[c, pl.ds(s * ROWS_PER_TILE, ROWS_PER_TILE)])


def _sc_agg(hin2, src3d, dst3d, zeros):
    mesh = plsc.VectorSubcoreMesh(core_axis_name="c", subcore_axis_name="s")
    return pl.kernel(
        _sc_agg_body,
        out_type=jax.ShapeDtypeStruct((NC, R_PAD, HHALF), jnp.float32),
        mesh=mesh,
        compiler_params=pltpu.CompilerParams(use_tc_tiling_on_sc=False),
        scratch_types=[
            pltpu.VMEM((CHUNKS, CHUNK), jnp.int32),
            pltpu.VMEM((CHUNKS, CHUNK), jnp.int32),
            pltpu.VMEM((NBUF, CHUNK, HHALF), jnp.float32),
            pltpu.VMEM_SHARED((R_PAD, HHALF), jnp.float32),
            pltpu.VMEM_SHARED((R_PAD, HHALF), jnp.float32),
            pltpu.SemaphoreType.DMA((NBUF,)),
            pltpu.SemaphoreType.DMA((NBUF,)),
        ],
    )(hin2, src3d, dst3d, zeros)


def kernel(x, edge_index, batch, struc, params):
    p0 = params['mlp0']
    W1f, c1 = _fold_bn(p0['W1'], p0['b1'], p0['bn1'])
    W2f, c2 = _fold_bn(p0['W2'], p0['b2'], p0['bn2'])
    w0 = (W1f[:D_FEAT], W1f[D_FEAT:], c1.reshape(1, HID),
          W2f, c2.reshape(1, HID))
    wl = {}
    for l in range(1, 5):
        pl_ = params['mlp%d' % l]
        W1f, c1 = _fold_bn(pl_['W1'], pl_['b1'], pl_['bn1'])
        W2f, c2 = _fold_bn(pl_['W2'], pl_['b2'], pl_['bn2'])
        W1e = jnp.zeros((HPAD, HID), jnp.float32).at[:HID + INFO_DIM].set(W1f)
        wl[l] = (W1e, c1.reshape(1, HID), W2f, c2.reshape(1, HID))
    lin = {}
    for l in range(5):
        lp = params['lin%d' % l]
        lin[l] = (lp['W'][:HID], lp['W'][HID:], lp['b'].reshape(1, TGT))

    x_pad = jnp.concatenate(
        [x, jnp.zeros((R_PAD - N_NODES, D_FEAT), jnp.float32)])
    batch_pad = jnp.concatenate(
        [batch, jnp.full((R_PAD - N_NODES,), -1, jnp.int32)])
    batch_col = batch_pad.reshape(R_PAD, 1)
    batch_row = batch_pad.reshape(GRID, 1, NB)

    src = edge_index[0]
    dst = edge_index[1]
    pad = E_PAD - N_EDGES
    src3d = jnp.concatenate(
        [src, jnp.full((pad,), PAD_ROW, jnp.int32)]).reshape(NS, CHUNKS, CHUNK)
    dst3d = jnp.concatenate(
        [dst, jnp.arange(pad, dtype=jnp.int32) % N_NODES]).reshape(NS, CHUNKS, CHUNK)
    zeros = jnp.zeros((R_PAD, HHALF), jnp.float32)

    rep = _rep_call(batch_col, struc)
    w1x, w1r, c1, w2, c2 = w0
    l0h, l0r, b0 = lin[0]
    l0e = jnp.concatenate(
        [l0h, l0r, b0, jnp.zeros((HPAD - HID - INFO_DIM - 1, TGT), jnp.float32)])
    out, hin = _layer0_call(x_pad, rep, batch_col, batch_row,
                            w1x, w1r, c1, w2, c2, l0e)

    for l in range(1, 5):
        agg2 = _sc_agg(hin, src3d, dst3d, zeros)
        w1, c1, w2, c2 = wl[l]
        lh, lr, bl = lin[l]
        outc, hin = _layer_call(hin, agg2, batch_col, batch_row, struc,
                                w1, c1, w2, c2, lh, lr, bl)
        out = out + outc
    return out
